# Initial kernel scaffold; baseline (speedup 1.0000x reference)
import jax
import jax.numpy as jnp
from jax.experimental import pallas as pl


def kernel(x, edge_index, batch, node_num, edge_num, start_node, gid, checkStatus, W1, b1, g1, be1, rm1, rv1, W2, b2, g2, be2, rm2, rv2, W3, b3, g3, be3, rm3, rv3, Wl1, bl1, Wl2, bl2):
    raise NotImplementedError("write your pallas kernel here")


# speedup vs baseline: 3.4303x; 3.4303x over previous
import functools

import jax
import jax.numpy as jnp
from jax import lax
from jax.experimental import pallas as pl
from jax.experimental.pallas import tpu as pltpu
from jax.experimental.pallas import tpu_sc as plsc

_N = 50000
_E = 800000
_G = 64
_BLK = 256
_NPAD = 50176
_NB = _NPAD // _BLK
_EPAD = 802816
_ER = _EPAD // 128
_EPT = _EPAD // 16
_CHUNK = 512
_CHUNKS = _EPT // _CHUNK
_RPT = _NPAD // 16

_HI = lax.Precision.HIGHEST


def _z(i):
  return jnp.zeros((), i.dtype)


def _make_seg_sum(ng):
  mesh = plsc.VectorSubcoreMesh(core_axis_name="c", subcore_axis_name="s")

  @functools.partial(
      pl.kernel,
      out_type=tuple(
          jax.ShapeDtypeStruct((_NPAD, 32), jnp.float32) for _ in range(ng)),
      mesh=mesh,
      scratch_types=[
          pltpu.VMEM((_CHUNK // 128, 128), jnp.int32),
          pltpu.VMEM((_CHUNK // 128, 128), jnp.int32),
          pltpu.VMEM((_CHUNK, 32), jnp.float32),
          pltpu.VMEM_SHARED((_NPAD, 32), jnp.float32),
          pltpu.SemaphoreType.DMA,
      ],
      compiler_params=pltpu.CompilerParams(use_tc_tiling_on_sc=False),
  )
  def seg(*refs):
    xs = refs[:ng]
    src2, dst2 = refs[ng], refs[ng + 1]
    outs = refs[ng + 2:2 * ng + 2]
    sbuf, dbuf, rows, acc, sem = refs[2 * ng + 2:]
    i32 = lambda v: jnp.asarray(v, jnp.int32)
    c = i32(lax.axis_index("c"))
    s = i32(lax.axis_index("s"))

    def zero_rows():
      def zb(i, carry):
        rows[i, pl.ds(0, 16)] = jnp.zeros((16,), jnp.float32)
        rows[i, pl.ds(16, 16)] = jnp.zeros((16,), jnp.float32)
        return carry

      lax.fori_loop(jnp.int32(0), jnp.int32(_CHUNK), zb, jnp.int32(0))

    nz_full = _RPT // _CHUNK
    for r in range(2):
      if r * 2 >= ng:
        continue
      zero_rows()
      for k in range(nz_full + 1):
        sz = _CHUNK if k < nz_full else _RPT - nz_full * _CHUNK
        if sz == 0:
          continue
        pltpu.sync_copy(rows.at[pl.ds(0, sz)],
                        acc.at[pl.ds(s * i32(_RPT) + i32(k * _CHUNK), sz)])
      plsc.subcore_barrier()

      for cc in range(2):
        g = r * 2 + cc
        if g >= ng:
          continue

        @pl.when(c == cc)
        def _proc(g=g):
          nj = _CHUNK // 128

          def chunk(ci, carry):
            row0 = s * i32(_EPT // 128) + ci * i32(nj)
            pltpu.sync_copy(src2.at[pl.ds(row0, nj)], sbuf)
            pltpu.sync_copy(dst2.at[pl.ds(row0, nj)], dbuf)
            cps = [
                pltpu.async_copy(
                    xs[g].at[sbuf.at[jnp.int32(j)]],
                    rows.at[pl.ds(j * 128, 128)], sem)
                for j in range(nj)
            ]
            for cp in cps:
              cp.wait()
            for j in range(nj):
              pltpu.sync_copy(rows.at[pl.ds(j * 128, 128)],
                              acc.at[dbuf.at[jnp.int32(j)]], add=True)
            return carry

          lax.fori_loop(jnp.int32(0), jnp.int32(_CHUNKS), chunk, jnp.int32(0))

      plsc.subcore_barrier()
      for cc in range(2):
        g = r * 2 + cc
        if g >= ng:
          continue

        @pl.when(c == cc)
        def _wr(g=g):
          pltpu.sync_copy(acc.at[pl.ds(s * i32(_RPT), _RPT)],
                          outs[g].at[pl.ds(s * i32(_RPT), _RPT)])

      plsc.subcore_barrier()

  return seg


def _gin_update(xg, ag, WT, b2d, relu):
  ng = len(xg)

  def body(*refs):
    x_refs = refs[:ng]
    a_refs = refs[ng:2 * ng]
    w_ref, b_ref = refs[2 * ng], refs[2 * ng + 1]
    o_refs = refs[2 * ng + 2:]
    i = pl.program_id(0)
    h = jnp.broadcast_to(b_ref[...], (_BLK, 128))
    for g in range(ng):
      h = h + jnp.dot(x_refs[g][...] + a_refs[g][...],
                      w_ref[g * 32:(g + 1) * 32, :],
                      precision=_HI, preferred_element_type=jnp.float32)
    if relu:
      h = jnp.maximum(h, 0.0)
    rid = i * _BLK + lax.broadcasted_iota(jnp.int32, (_BLK, 1), 0)
    h = jnp.where(rid < _N, h, 0.0)
    for go in range(4):
      o_refs[go][...] = h[:, go * 32:(go + 1) * 32]

  grp_spec = pl.BlockSpec((_BLK, 32), lambda i: (i, _z(i)))
  return pl.pallas_call(
      body,
      grid=(_NB,),
      in_specs=[grp_spec] * (2 * ng) + [
          pl.BlockSpec((128, 128), lambda i: (_z(i), _z(i))),
          pl.BlockSpec((1, 128), lambda i: (_z(i), _z(i))),
      ],
      out_specs=[grp_spec] * 4,
      out_shape=tuple(
          jax.ShapeDtypeStruct((_NPAD, 32), jnp.float32) for _ in range(4)),
  )(*xg, *ag, WT, b2d)


def _final(xg, ag, WT3, b3_2d, Wl1T, bl1_2d, Wl2T, bl2_2d, bt3):
  def body(*refs):
    x_refs = refs[:4]
    a_refs = refs[4:8]
    (w_ref, b_ref, wl1_ref, bl1_ref, wl2_ref, bl2_ref, bt_ref, o_ref,
     sums, cnt) = refs[8:]
    i = pl.program_id(0)

    @pl.when(i == 0)
    def _init():
      sums[...] = jnp.zeros_like(sums)
      cnt[...] = jnp.zeros_like(cnt)

    h3 = jnp.broadcast_to(b_ref[...], (_BLK, 128))
    for g in range(4):
      h3 = h3 + jnp.dot(x_refs[g][...] + a_refs[g][...],
                        w_ref[g * 32:(g + 1) * 32, :],
                        precision=_HI, preferred_element_type=jnp.float32)
    h4 = jnp.maximum(
        jnp.dot(h3, wl1_ref[...], precision=_HI,
                preferred_element_type=jnp.float32) + bl1_ref[...], 0.0)
    btv = bt_ref[0, 0]
    oh = (btv[:, None] == lax.broadcasted_iota(jnp.int32, (_BLK, _G), 1)
          ).astype(jnp.float32)
    sums[...] += lax.dot_general(oh, h4, (((0,), (0,)), ((), ())),
                                 precision=_HI,
                                 preferred_element_type=jnp.float32)
    cnt[...] += lax.dot_general(oh, jnp.ones((_BLK, 128), jnp.float32),
                                (((0,), (0,)), ((), ())),
                                precision=_HI,
                                preferred_element_type=jnp.float32)

    @pl.when(i == _NB - 1)
    def _fin():
      pooled = sums[...] / jnp.maximum(cnt[...], 1.0)
      t = jnp.maximum(
          jnp.dot(pooled, wl1_ref[...], precision=_HI,
                  preferred_element_type=jnp.float32) + bl1_ref[...], 0.0)
      o_ref[...] = jnp.dot(t, wl2_ref[...], precision=_HI,
                           preferred_element_type=jnp.float32) + bl2_ref[...]

  grp_spec = pl.BlockSpec((_BLK, 32), lambda i: (i, _z(i)))
  full_spec = pl.BlockSpec((128, 128), lambda i: (_z(i), _z(i)))
  bias_spec = pl.BlockSpec((1, 128), lambda i: (_z(i), _z(i)))
  return pl.pallas_call(
      body,
      grid=(_NB,),
      in_specs=[grp_spec] * 8 + [
          full_spec, bias_spec, full_spec, bias_spec, full_spec, bias_spec,
          pl.BlockSpec((1, 1, _BLK), lambda i: (i, _z(i), _z(i))),
      ],
      out_specs=pl.BlockSpec((_G, 128), lambda i: (_z(i), _z(i))),
      out_shape=jax.ShapeDtypeStruct((_G, 128), jnp.float32),
      scratch_shapes=[
          pltpu.VMEM((_G, 128), jnp.float32),
          pltpu.VMEM((_G, 128), jnp.float32),
      ],
  )(*xg, *ag, WT3, b3_2d, Wl1T, bl1_2d, Wl2T, bl2_2d, bt3)


def kernel(x, edge_index, batch, node_num, edge_num, start_node, gid,
           checkStatus, W1, b1, g1, be1, rm1, rv1, W2, b2, g2, be2, rm2, rv2,
           W3, b3, g3, be3, rm3, rv3, Wl1, bl1, Wl2, bl2):
  f32 = jnp.float32
  x = x.astype(f32)

  def fold(W, b, gm, be, rm, rv):
    inv = gm / jnp.sqrt(rv + 1e-5)
    return W * inv[:, None], (b - rm) * inv + be

  W1f, b1f = fold(W1, b1, g1, be1, rm1, rv1)
  W2f, b2f = fold(W2, b2, g2, be2, rm2, rv2)
  W3f, b3f = fold(W3, b3, g3, be3, rm3, rv3)

  in_ch = x.shape[1]
  ng1 = -(-in_ch // 32)
  WT1 = jnp.zeros((128, 128), f32).at[:in_ch, :].set(W1f.T)
  WT2 = W2f.T.astype(f32)
  WT3 = W3f.T.astype(f32)
  Wl1T = Wl1.T.astype(f32)
  Wl2T = Wl2.T.astype(f32)

  xp = jnp.zeros((_NPAD, 128), f32).at[:_N, :in_ch].set(x)
  xg = tuple(xp[:, 32 * g:32 * (g + 1)] for g in range(ng1))

  src = edge_index[0].astype(jnp.int32)
  dst = edge_index[1].astype(jnp.int32)
  padi = _N + (jnp.arange(_EPAD - _E, dtype=jnp.int32) % (_NPAD - _N))
  src2 = jnp.concatenate([src, padi]).reshape(_ER, 128)
  dst2 = jnp.concatenate([dst, padi]).reshape(_ER, 128)

  bt3 = (jnp.full((_NPAD,), _G, jnp.int32)
         .at[:_N].set(batch.astype(jnp.int32))
         .reshape(_NB, 1, _BLK))

  seg_a = _make_seg_sum(ng1)
  seg_b = _make_seg_sum(4)

  agg1 = seg_a(*xg, src2, dst2)
  h1 = _gin_update(xg, agg1, WT1, b1f.reshape(1, 128), relu=True)
  agg2 = seg_b(*h1, src2, dst2)
  h2 = _gin_update(h1, agg2, WT2, b2f.reshape(1, 128), relu=True)
  agg3 = seg_b(*h2, src2, dst2)
  return _final(h2, agg3, WT3, b3f.reshape(1, 128), Wl1T,
                bl1.reshape(1, 128).astype(f32), Wl2T,
                bl2.reshape(1, 128).astype(f32), bt3)


# speedup vs baseline: 4.7939x; 1.3975x over previous
import functools

import jax
import jax.numpy as jnp
from jax import lax
from jax.experimental import pallas as pl
from jax.experimental.pallas import tpu as pltpu
from jax.experimental.pallas import tpu_sc as plsc

_N = 50000
_E = 800000
_G = 64
_BLK = 256
_NPAD = 50176
_NB = _NPAD // _BLK
_EPAD = 802816
_ER = _EPAD // 128
_EPT = _EPAD // 16
_CHUNK = 256
_NJ = _CHUNK // 128
_NCH = _EPT // _CHUNK
_KB = _NCH // 2
_RPT = _NPAD // 16

_HI = lax.Precision.HIGHEST


def _z(i):
  return jnp.zeros((), i.dtype)


def _make_seg_sum(ng):
  mesh = plsc.VectorSubcoreMesh(core_axis_name="c", subcore_axis_name="s")

  @functools.partial(
      pl.kernel,
      out_type=tuple(
          jax.ShapeDtypeStruct((_NPAD, 32), jnp.float32) for _ in range(ng)),
      mesh=mesh,
      scratch_types=[
          pltpu.VMEM((2, _NJ, 128), jnp.int32),
          pltpu.VMEM((3, _NJ, 128), jnp.int32),
          pltpu.VMEM((2, _CHUNK, 32), jnp.float32),
          pltpu.VMEM_SHARED((_NPAD, 32), jnp.float32),
          pltpu.SemaphoreType.DMA,
          pltpu.SemaphoreType.DMA,
          pltpu.SemaphoreType.DMA,
      ],
      compiler_params=pltpu.CompilerParams(use_tc_tiling_on_sc=False),
  )
  def seg(*refs):
    xs = refs[:ng]
    src2, dst2 = refs[ng], refs[ng + 1]
    outs = refs[ng + 2:2 * ng + 2]
    sbuf, dbuf, rows, acc, sem_i, sem_g, sem_s = refs[2 * ng + 2:]
    i32 = lambda v: jnp.asarray(v, jnp.int32)
    c = i32(lax.axis_index("c"))
    s = i32(lax.axis_index("s"))
    z = jnp.int32(0)

    def zero_rows():
      def zb(i, carry):
        rows[z, i, pl.ds(0, 16)] = jnp.zeros((16,), jnp.float32)
        rows[z, i, pl.ds(16, 16)] = jnp.zeros((16,), jnp.float32)
        return carry

      lax.fori_loop(jnp.int32(0), jnp.int32(_CHUNK), zb, jnp.int32(0))

    def load_idx(ci, ib, db, sync):
      r0 = s * i32(_EPT // 128) + ci * i32(_NJ)
      if sync:
        pltpu.sync_copy(src2.at[pl.ds(r0, _NJ)], sbuf.at[ib])
        pltpu.sync_copy(dst2.at[pl.ds(r0, _NJ)], dbuf.at[db])
      else:
        pltpu.async_copy(src2.at[pl.ds(r0, _NJ)], sbuf.at[ib], sem_i)
        pltpu.async_copy(dst2.at[pl.ds(r0, _NJ)], dbuf.at[db], sem_i)

    def wait_idx():
      for _ in range(2):
        pltpu.make_async_copy(src2.at[pl.ds(z, _NJ)], sbuf.at[z],
                              sem_i).wait()

    def fire_gathers(xr, ib, rp):
      for j in range(_NJ):
        pltpu.async_copy(xr.at[sbuf.at[ib, jnp.int32(j)]],
                         rows.at[jnp.int32(rp), pl.ds(j * 128, 128)], sem_g)

    def wait_gathers(xr):
      for _ in range(_NJ):
        pltpu.make_async_copy(xr.at[sbuf.at[z, z]],
                              rows.at[z, pl.ds(0, 128)], sem_g).wait()

    def fire_scatters(db, rp):
      for j in range(_NJ):
        pltpu.async_copy(rows.at[jnp.int32(rp), pl.ds(j * 128, 128)],
                         acc.at[dbuf.at[db, jnp.int32(j)]], sem_s, add=True)

    def wait_scatters():
      for _ in range(_NJ):
        pltpu.make_async_copy(rows.at[z, pl.ds(0, 128)],
                              acc.at[dbuf.at[z, z]], sem_s).wait()

    nz_full = _RPT // _CHUNK
    for r in range(2):
      if r * 2 >= ng:
        continue
      zero_rows()
      for k in range(nz_full + 1):
        sz = _CHUNK if k < nz_full else _RPT - nz_full * _CHUNK
        if sz == 0:
          continue
        pltpu.sync_copy(rows.at[z, pl.ds(0, sz)],
                        acc.at[pl.ds(s * i32(_RPT) + i32(k * _CHUNK), sz)])
      plsc.subcore_barrier()

      for cc in range(2):
        g = r * 2 + cc
        if g >= ng:
          continue

        @pl.when(c == cc)
        def _proc(g=g):
          xr = xs[g]
          load_idx(z, z, z, sync=True)
          fire_gathers(xr, z, 0)
          load_idx(jnp.int32(1), jnp.int32(1), jnp.int32(1), sync=False)

          def body(k, carry):
            for p in range(2):
              ci = k * i32(2) + i32(p)
              if p == 0:
                @pl.when(k > 0)
                def _():
                  wait_scatters()
              else:
                wait_scatters()
              if p == 0:
                wait_idx()
                fire_gathers(xr, jnp.int32(1), 1)
              else:
                @pl.when(k < _KB - 1)
                def _():
                  wait_idx()
                  fire_gathers(xr, z, 0)
              wait_gathers(xr)
              @pl.when(k < _KB - 1)
              def _():
                load_idx(ci + i32(2), i32(p), lax.rem(ci + i32(2), i32(3)),
                         sync=False)
              fire_scatters(lax.rem(ci, i32(3)), p)
            return carry

          lax.fori_loop(jnp.int32(0), jnp.int32(_KB), body, jnp.int32(0))
          wait_scatters()

      plsc.subcore_barrier()
      for cc in range(2):
        g = r * 2 + cc
        if g >= ng:
          continue

        @pl.when(c == cc)
        def _wr(g=g):
          pltpu.sync_copy(acc.at[pl.ds(s * i32(_RPT), _RPT)],
                          outs[g].at[pl.ds(s * i32(_RPT), _RPT)])

      plsc.subcore_barrier()

  return seg


def _gin_update(xg, ag, WT, b2d, relu):
  ng = len(xg)

  def body(*refs):
    x_refs = refs[:ng]
    a_refs = refs[ng:2 * ng]
    w_ref, b_ref = refs[2 * ng], refs[2 * ng + 1]
    o_refs = refs[2 * ng + 2:]
    i = pl.program_id(0)
    h = jnp.broadcast_to(b_ref[...], (_BLK, 128))
    for g in range(ng):
      h = h + jnp.dot(x_refs[g][...] + a_refs[g][...],
                      w_ref[g * 32:(g + 1) * 32, :],
                      precision=_HI, preferred_element_type=jnp.float32)
    if relu:
      h = jnp.maximum(h, 0.0)
    rid = i * _BLK + lax.broadcasted_iota(jnp.int32, (_BLK, 1), 0)
    h = jnp.where(rid < _N, h, 0.0)
    for go in range(4):
      o_refs[go][...] = h[:, go * 32:(go + 1) * 32]

  grp_spec = pl.BlockSpec((_BLK, 32), lambda i: (i, _z(i)))
  return pl.pallas_call(
      body,
      grid=(_NB,),
      in_specs=[grp_spec] * (2 * ng) + [
          pl.BlockSpec((128, 128), lambda i: (_z(i), _z(i))),
          pl.BlockSpec((1, 128), lambda i: (_z(i), _z(i))),
      ],
      out_specs=[grp_spec] * 4,
      out_shape=tuple(
          jax.ShapeDtypeStruct((_NPAD, 32), jnp.float32) for _ in range(4)),
  )(*xg, *ag, WT, b2d)


def _final(xg, ag, WT3, b3_2d, Wl1T, bl1_2d, Wl2T, bl2_2d, bt3):
  def body(*refs):
    x_refs = refs[:4]
    a_refs = refs[4:8]
    (w_ref, b_ref, wl1_ref, bl1_ref, wl2_ref, bl2_ref, bt_ref, o_ref,
     sums, cnt) = refs[8:]
    i = pl.program_id(0)

    @pl.when(i == 0)
    def _init():
      sums[...] = jnp.zeros_like(sums)
      cnt[...] = jnp.zeros_like(cnt)

    h3 = jnp.broadcast_to(b_ref[...], (_BLK, 128))
    for g in range(4):
      h3 = h3 + jnp.dot(x_refs[g][...] + a_refs[g][...],
                        w_ref[g * 32:(g + 1) * 32, :],
                        precision=_HI, preferred_element_type=jnp.float32)
    h4 = jnp.maximum(
        jnp.dot(h3, wl1_ref[...], precision=_HI,
                preferred_element_type=jnp.float32) + bl1_ref[...], 0.0)
    btv = bt_ref[0, 0]
    oh = (btv[:, None] == lax.broadcasted_iota(jnp.int32, (_BLK, _G), 1)
          ).astype(jnp.float32)
    sums[...] += lax.dot_general(oh, h4, (((0,), (0,)), ((), ())),
                                 precision=_HI,
                                 preferred_element_type=jnp.float32)
    cnt[...] += lax.dot_general(oh, jnp.ones((_BLK, 128), jnp.float32),
                                (((0,), (0,)), ((), ())),
                                precision=_HI,
                                preferred_element_type=jnp.float32)

    @pl.when(i == _NB - 1)
    def _fin():
      pooled = sums[...] / jnp.maximum(cnt[...], 1.0)
      t = jnp.maximum(
          jnp.dot(pooled, wl1_ref[...], precision=_HI,
                  preferred_element_type=jnp.float32) + bl1_ref[...], 0.0)
      o_ref[...] = jnp.dot(t, wl2_ref[...], precision=_HI,
                           preferred_element_type=jnp.float32) + bl2_ref[...]

  grp_spec = pl.BlockSpec((_BLK, 32), lambda i: (i, _z(i)))
  full_spec = pl.BlockSpec((128, 128), lambda i: (_z(i), _z(i)))
  bias_spec = pl.BlockSpec((1, 128), lambda i: (_z(i), _z(i)))
  return pl.pallas_call(
      body,
      grid=(_NB,),
      in_specs=[grp_spec] * 8 + [
          full_spec, bias_spec, full_spec, bias_spec, full_spec, bias_spec,
          pl.BlockSpec((1, 1, _BLK), lambda i: (i, _z(i), _z(i))),
      ],
      out_specs=pl.BlockSpec((_G, 128), lambda i: (_z(i), _z(i))),
      out_shape=jax.ShapeDtypeStruct((_G, 128), jnp.float32),
      scratch_shapes=[
          pltpu.VMEM((_G, 128), jnp.float32),
          pltpu.VMEM((_G, 128), jnp.float32),
      ],
  )(*xg, *ag, WT3, b3_2d, Wl1T, bl1_2d, Wl2T, bl2_2d, bt3)


def kernel(x, edge_index, batch, node_num, edge_num, start_node, gid,
           checkStatus, W1, b1, g1, be1, rm1, rv1, W2, b2, g2, be2, rm2, rv2,
           W3, b3, g3, be3, rm3, rv3, Wl1, bl1, Wl2, bl2):
  f32 = jnp.float32
  x = x.astype(f32)

  def fold(W, b, gm, be, rm, rv):
    inv = gm / jnp.sqrt(rv + 1e-5)
    return W * inv[:, None], (b - rm) * inv + be

  W1f, b1f = fold(W1, b1, g1, be1, rm1, rv1)
  W2f, b2f = fold(W2, b2, g2, be2, rm2, rv2)
  W3f, b3f = fold(W3, b3, g3, be3, rm3, rv3)

  in_ch = x.shape[1]
  ng1 = -(-in_ch // 32)
  WT1 = jnp.zeros((128, 128), f32).at[:in_ch, :].set(W1f.T)
  WT2 = W2f.T.astype(f32)
  WT3 = W3f.T.astype(f32)
  Wl1T = Wl1.T.astype(f32)
  Wl2T = Wl2.T.astype(f32)

  xp = jnp.zeros((_NPAD, 128), f32).at[:_N, :in_ch].set(x)
  xg = tuple(xp[:, 32 * g:32 * (g + 1)] for g in range(ng1))

  src = edge_index[0].astype(jnp.int32)
  dst = edge_index[1].astype(jnp.int32)
  padi = _N + (jnp.arange(_EPAD - _E, dtype=jnp.int32) % (_NPAD - _N))
  src2 = jnp.concatenate([src, padi]).reshape(_ER, 128)
  dst2 = jnp.concatenate([dst, padi]).reshape(_ER, 128)

  bt3 = (jnp.full((_NPAD,), _G, jnp.int32)
         .at[:_N].set(batch.astype(jnp.int32))
         .reshape(_NB, 1, _BLK))

  seg_a = _make_seg_sum(ng1)
  seg_b = _make_seg_sum(4)

  agg1 = seg_a(*xg, src2, dst2)
  h1 = _gin_update(xg, agg1, WT1, b1f.reshape(1, 128), relu=True)
  agg2 = seg_b(*h1, src2, dst2)
  h2 = _gin_update(h1, agg2, WT2, b2f.reshape(1, 128), relu=True)
  agg3 = seg_b(*h2, src2, dst2)
  return _final(h2, agg3, WT3, b3f.reshape(1, 128), Wl1T,
                bl1.reshape(1, 128).astype(f32), Wl2T,
                bl2.reshape(1, 128).astype(f32), bt3)
